# Initial kernel scaffold; baseline (speedup 1.0000x reference)
import jax
import jax.numpy as jnp
from jax.experimental import pallas as pl


def kernel(x_paper, x_author, edge_writes, edge_cites, edge_written_by, W1_writes, b1_writes, W1_cites, b1_cites, W1_written_by, b1_written_by, W2_writes, b2_writes, W2_cites, b2_cites, W2_written_by, b2_written_by):
    raise NotImplementedError("write your pallas kernel here")


# speedup vs baseline: 4.4630x; 4.4630x over previous
import functools

import jax
import jax.numpy as jnp
from jax import lax
from jax.experimental import pallas as pl
from jax.experimental.pallas import tpu as pltpu
from jax.experimental.pallas import tpu_sc as plsc

N = 10000
NPAD = 10240
D_IN = 128
HIDDEN = 128
OUT = 64
E = 160000
CH = 128
NCHUNK = E // CH
NC = 2
NS = 16
NW = NC * NS
TPW = (NCHUNK + NW - 1) // NW
RPT = NPAD // NS


def _zero16():
    return jnp.zeros((16,), jnp.float32)

def _worker_id():
    return lax.axis_index("s") * NC + lax.axis_index("c")


@functools.partial(
    pl.kernel,
    out_type=jax.ShapeDtypeStruct((NW, 6, NPAD), jnp.float32),
    mesh=plsc.VectorSubcoreMesh(core_axis_name="c", subcore_axis_name="s"),
    scratch_types=[
        pltpu.VMEM((CH,), jnp.int32),
        [pltpu.VMEM((NPAD,), jnp.float32) for _ in range(6)],
    ],
    compiler_params=pltpu.CompilerParams(needs_layout_passes=False),
)
def _degrees_sc(e0, e1, e2, e3, e4, e5, out, idx_v, hists):
    wid = _worker_id()

    zero16 = _zero16()
    one16 = jnp.ones((16,), jnp.float32)

    def zbody(i, _):
        for h in hists:
            h[pl.ds(i * 16, 16)] = zero16
        return 0
    lax.fori_loop(0, NPAD // 16, zbody, 0)

    for arr, hist in zip((e0, e1, e2, e3, e4, e5), hists):
        def body(t, _, arr=arr, hist=hist):
            k = wid + t * NW

            @pl.when(k < NCHUNK)
            def _():
                pltpu.sync_copy(arr.at[pl.ds(k * CH, CH)], idx_v)
                for j in range(CH // 16):
                    idx16 = idx_v[pl.ds(j * 16, 16)]
                    plsc.addupdate_scatter(hist, [idx16], one16)
            return 0
        lax.fori_loop(0, TPW, body, 0)

    for r, hist in enumerate(hists):
        pltpu.sync_copy(hist, out.at[wid, r])


def _make_agg(D):
    @functools.partial(
        pl.kernel,
        out_type=jax.ShapeDtypeStruct((NC, NPAD, D), jnp.float32),
        mesh=plsc.VectorSubcoreMesh(core_axis_name="c", subcore_axis_name="s"),
        scratch_types=[
            pltpu.VMEM((CH,), jnp.int32),
            pltpu.VMEM((CH,), jnp.int32),
            pltpu.VMEM((CH, D), jnp.float32),
            pltpu.VMEM((CH, D), jnp.float32),
            pltpu.VMEM_SHARED((NPAD, D), jnp.float32),
            pltpu.SemaphoreType.DMA,
        ],
        compiler_params=pltpu.CompilerParams(use_tc_tiling_on_sc=False),
    )
    def agg(y, src, dst, out, src_v, dst_v, rows_v, stage_v, acc_sh, sem):
        c = lax.axis_index("c")
        s = lax.axis_index("s")
        wid = s * NC + c

        zero16 = _zero16()

        def zbody(i, _):
            for j in range(D // 16):
                stage_v[i, pl.ds(j * 16, 16)] = zero16
            return 0
        lax.fori_loop(0, CH, zbody, 0)
        for q in range(RPT // CH):
            pltpu.sync_copy(stage_v, acc_sh.at[pl.ds(s * RPT + q * CH, CH)])
        plsc.subcore_barrier()

        def body(t, _):
            k = wid + t * NW

            @pl.when(k < NCHUNK)
            def _():
                pltpu.sync_copy(src.at[pl.ds(k * CH, CH)], src_v)
                pltpu.sync_copy(dst.at[pl.ds(k * CH, CH)], dst_v)
                pltpu.async_copy(y.at[src_v], rows_v, sem).wait()
                pltpu.sync_copy(rows_v, acc_sh.at[dst_v], add=True)
            return 0
        lax.fori_loop(0, TPW, body, 0)
        plsc.subcore_barrier()

        for q in range(RPT // CH):
            off = s * RPT + q * CH
            pltpu.sync_copy(acc_sh.at[pl.ds(off, CH)], stage_v)
            pltpu.sync_copy(stage_v, out.at[c, pl.ds(off, CH)])

    return agg


_agg_h = _make_agg(HIDDEN)
_agg_o = _make_agg(OUT)


def _degsum_body(dp_ref, out_ref):
    dg = jnp.sum(dp_ref[...], axis=0)
    out_ref[...] = jnp.where(dg > 0, lax.rsqrt(jnp.maximum(dg, 1.0)), 0.0)


def _norms_tc(degparts):
    return pl.pallas_call(
        _degsum_body,
        grid=(NPAD // 128,),
        in_specs=[pl.BlockSpec((NW, 6, 128), lambda i: (0, 0, i))],
        out_specs=pl.BlockSpec((6, 128), lambda i: (0, i)),
        out_shape=jax.ShapeDtypeStruct((6, NPAD), jnp.float32),
    )(degparts)


def _proj_body(x_ref, n_ref, w_ref, o_ref):
    o_ref[...] = jnp.dot(x_ref[...] * n_ref[...], w_ref[...],
                         preferred_element_type=jnp.float32)


def _proj_tc(x, ncol, W):
    H = W.shape[1]
    return pl.pallas_call(
        _proj_body,
        grid=(NPAD // 128,),
        in_specs=[
            pl.BlockSpec((128, 128), lambda i: (i, 0)),
            pl.BlockSpec((128, 1), lambda i: (i, 0)),
            pl.BlockSpec((128, H), lambda i: (0, 0)),
        ],
        out_specs=pl.BlockSpec((128, H), lambda i: (i, 0)),
        out_shape=jax.ShapeDtypeStruct((NPAD, H), jnp.float32),
    )(x, ncol, W)


def _comb2_body(relu, a_ref, c_ref, na_ref, nc_ref, ba_ref, bc_ref, o_ref):
    v = ((a_ref[0] + a_ref[1]) * na_ref[...]
         + (c_ref[0] + c_ref[1]) * nc_ref[...]
         + ba_ref[...] + bc_ref[...])
    o_ref[...] = jnp.maximum(v, 0.0) if relu else v


def _comb2_tc(agg_a, agg_c, n_a, n_c, b_a, b_c, relu):
    D = agg_a.shape[-1]
    return pl.pallas_call(
        functools.partial(_comb2_body, relu),
        grid=(NPAD // 128,),
        in_specs=[
            pl.BlockSpec((NC, 128, D), lambda i: (0, i, 0)),
            pl.BlockSpec((NC, 128, D), lambda i: (0, i, 0)),
            pl.BlockSpec((128, 1), lambda i: (i, 0)),
            pl.BlockSpec((128, 1), lambda i: (i, 0)),
            pl.BlockSpec((1, D), lambda i: (0, 0)),
            pl.BlockSpec((1, D), lambda i: (0, 0)),
        ],
        out_specs=pl.BlockSpec((128, D), lambda i: (i, 0)),
        out_shape=jax.ShapeDtypeStruct((NPAD, D), jnp.float32),
    )(agg_a, agg_c, n_a, n_c, b_a.reshape(1, D), b_c.reshape(1, D))


def _comb1_body(relu, a_ref, na_ref, ba_ref, o_ref):
    v = (a_ref[0] + a_ref[1]) * na_ref[...] + ba_ref[...]
    o_ref[...] = jnp.maximum(v, 0.0) if relu else v


def _comb1_tc(agg_a, n_a, b_a, relu):
    D = agg_a.shape[-1]
    return pl.pallas_call(
        functools.partial(_comb1_body, relu),
        grid=(NPAD // 128,),
        in_specs=[
            pl.BlockSpec((NC, 128, D), lambda i: (0, i, 0)),
            pl.BlockSpec((128, 1), lambda i: (i, 0)),
            pl.BlockSpec((1, D), lambda i: (0, 0)),
        ],
        out_specs=pl.BlockSpec((128, D), lambda i: (i, 0)),
        out_shape=jax.ShapeDtypeStruct((NPAD, D), jnp.float32),
    )(agg_a, n_a, b_a.reshape(1, D))


def kernel(x_paper, x_author, edge_writes, edge_cites, edge_written_by,
           W1_writes, b1_writes, W1_cites, b1_cites, W1_written_by, b1_written_by,
           W2_writes, b2_writes, W2_cites, b2_cites, W2_written_by, b2_written_by):
    pad = ((0, NPAD - N), (0, 0))
    xp = jnp.pad(x_paper, pad)
    xa = jnp.pad(x_author, pad)
    ew_s, ew_d = edge_writes[0], edge_writes[1]
    ec_s, ec_d = edge_cites[0], edge_cites[1]
    eb_s, eb_d = edge_written_by[0], edge_written_by[1]

    degparts = _degrees_sc(ew_s, ew_d, ec_s, ec_d, eb_s, eb_d)
    norms = _norms_tc(degparts)
    nsw = norms[0].reshape(NPAD, 1)
    ndw = norms[1].reshape(NPAD, 1)
    nsc = norms[2].reshape(NPAD, 1)
    ndc = norms[3].reshape(NPAD, 1)
    nsb = norms[4].reshape(NPAD, 1)
    ndb = norms[5].reshape(NPAD, 1)

    y1a = _proj_tc(xa, nsw, W1_writes)
    y1c = _proj_tc(xp, nsc, W1_cites)
    y1b = _proj_tc(xp, nsb, W1_written_by)
    s1w = _agg_h(y1a, ew_s, ew_d)
    s1c = _agg_h(y1c, ec_s, ec_d)
    s1b = _agg_h(y1b, eb_s, eb_d)
    h_paper = _comb2_tc(s1w, s1c, ndw, ndc, b1_writes, b1_cites, relu=True)
    h_author = _comb1_tc(s1b, ndb, b1_written_by, relu=True)

    y2a = _proj_tc(h_author, nsw, W2_writes)
    y2c = _proj_tc(h_paper, nsc, W2_cites)
    y2b = _proj_tc(h_paper, nsb, W2_written_by)
    s2w = _agg_o(y2a, ew_s, ew_d)
    s2c = _agg_o(y2c, ec_s, ec_d)
    s2b = _agg_o(y2b, eb_s, eb_d)
    out_paper = _comb2_tc(s2w, s2c, ndw, ndc, b2_writes, b2_cites, relu=False)
    out_author = _comb1_tc(s2b, ndb, b2_written_by, relu=False)

    return out_paper[:N], out_author[:N]
